# baseline (speedup 1.0000x reference)
import functools

import jax
import jax.numpy as jnp
from jax import lax
from jax.experimental import pallas as pl
from jax.experimental.pallas import tpu as pltpu
from jax.experimental.pallas import tpu_sc as plsc

_NW = 32
_PAD = 16384


def _mv_body(x_ref, w_ref, b_ref, out_ref):
    out_ref[...] = (
        jnp.sum(x_ref[...] * w_ref[...], axis=1, keepdims=True) + b_ref[0, 0]
    )


def _matvec(X, Wt, b2):
    N, D = X.shape
    BLK = 16384
    return pl.pallas_call(
        _mv_body,
        grid=(N // BLK,),
        in_specs=[
            pl.BlockSpec((BLK, D), lambda i: (i, 0)),
            pl.BlockSpec((1, D), lambda i: (0, 0)),
            pl.BlockSpec(memory_space=pltpu.SMEM),
        ],
        out_specs=pl.BlockSpec((BLK, 1), lambda i: (i, 0)),
        out_shape=jax.ShapeDtypeStruct((N, 1), jnp.float32),
    )(X, Wt, b2)


def _sc_fuse(bags2, mask2, nsent):
    nw, CHUNK = bags2.shape
    mesh = plsc.VectorSubcoreMesh(core_axis_name="c", subcore_axis_name="s")

    @functools.partial(
        pl.kernel,
        out_type=jax.ShapeDtypeStruct((nw, CHUNK), jnp.int32),
        mesh=mesh,
        scratch_types=[
            pltpu.VMEM((CHUNK,), jnp.int32),
            pltpu.VMEM((CHUNK,), jnp.int32),
        ],
    )
    def k(bags_hbm, mask_hbm, out_hbm, idx_v, msk_v):
        wid = lax.axis_index("s") * 2 + lax.axis_index("c")
        pltpu.sync_copy(bags_hbm.at[wid], idx_v)
        pltpu.sync_copy(mask_hbm.at[wid], msk_v)
        lane = lax.iota(jnp.int32, 16)

        def body(j, carry):
            off = j * 64
            for u in range(4):
                o = off + u * 16
                idx = idx_v[pl.ds(o, 16)]
                m = msk_v[pl.ds(o, 16)]
                pos = jnp.bitwise_and(o + lane, _PAD - 1)
                idx_v[pl.ds(o, 16)] = jnp.where(m != 0, nsent + pos, idx)
            return carry

        lax.fori_loop(0, CHUNK // 64, body, 0)
        pltpu.sync_copy(idx_v, out_hbm.at[wid])

    return k(bags2, mask2)


def _sc_gather_max(table, fused2, L):
    nw, CHUNK = fused2.shape
    bags_per_tile = CHUNK // L
    vregs_per_bag = L // 16
    mesh = plsc.VectorSubcoreMesh(core_axis_name="c", subcore_axis_name="s")
    oshape = jax.ShapeDtypeStruct((nw, bags_per_tile), jnp.float32)

    @functools.partial(
        pl.kernel,
        out_type=(oshape, oshape),
        mesh=mesh,
        scratch_types=[
            pltpu.VMEM((CHUNK,), jnp.int32),
            pltpu.VMEM((CHUNK,), jnp.float32),
            pltpu.VMEM((bags_per_tile,), jnp.float32),
            pltpu.VMEM((bags_per_tile,), jnp.float32),
            pltpu.SemaphoreType.DMA,
        ],
    )
    def k(table_hbm, fused_hbm, o0_hbm, o1_hbm, idx_v, vals_v, o0_v, o1_v, sem):
        wid = lax.axis_index("s") * 2 + lax.axis_index("c")
        pltpu.sync_copy(fused_hbm.at[wid], idx_v)
        pltpu.async_copy(table_hbm.at[idx_v], vals_v, sem).wait()
        neg = jnp.full((16,), -jnp.inf, dtype=jnp.float32)
        lane = lax.iota(jnp.int32, 16)

        dnums = lax.GatherDimensionNumbers(
            offset_dims=(), collapsed_slice_dims=(0,), start_index_map=(0,)
        )

        def vmax16(a):
            for s in (1, 2, 4, 8):
                perm = jnp.bitwise_xor(lane, s)
                shuf = lax.gather(
                    a, perm[:, None], dnums, (1,),
                    mode=lax.GatherScatterMode.PROMISE_IN_BOUNDS,
                )
                a = jnp.maximum(a, shuf)
            return a

        def group_body(g, carry):
            res = neg
            for k2 in range(16):
                off0 = (g * 16 + k2) * L
                acc = vals_v[pl.ds(off0, 16)]
                for i in range(1, vregs_per_bag):
                    acc = jnp.maximum(acc, vals_v[pl.ds(off0 + i * 16, 16)])
                res = jnp.where(lane == k2, vmax16(acc), res)
            t = jnp.exp(-jnp.abs(res))
            c = 1.0 + t
            y = 0.7 * t
            for _ in range(4):
                y = y + c * jnp.exp(-y) - 1.0
            sp = jnp.maximum(res, 0.0) + y
            o0_v[pl.ds(g * 16, 16)] = -sp
            o1_v[pl.ds(g * 16, 16)] = res - sp
            return carry

        lax.fori_loop(0, bags_per_tile // 16, group_body, 0)
        pltpu.sync_copy(o0_v, o0_hbm.at[wid])
        pltpu.sync_copy(o1_v, o1_hbm.at[wid])

    return k(table, fused2)


def kernel(X, bags, padding_mask, W, b):
    N, D = X.shape
    B, L = bags.shape
    logits = _matvec(X, W.reshape(1, D), b.reshape(1, 1))
    table = jnp.concatenate(
        [logits.reshape(N), jnp.full((_PAD,), -jnp.inf, jnp.float32)]
    )
    bags2 = bags.astype(jnp.int32).reshape(_NW, (B * L) // _NW)
    mask2 = padding_mask.astype(jnp.int32).reshape(_NW, (B * L) // _NW)
    fused2 = _sc_fuse(bags2, mask2, N)
    o0, o1 = _sc_gather_max(table, fused2, L)
    return jnp.stack([o0.reshape(B), o1.reshape(B)], axis=-1)


# speedup vs baseline: 1.0003x; 1.0003x over previous
import functools

import jax
import jax.numpy as jnp
from jax import lax
from jax.experimental import pallas as pl
from jax.experimental.pallas import tpu as pltpu
from jax.experimental.pallas import tpu_sc as plsc

_NW = 32
_PAD = 16384


def _mv_body(x_ref, w_ref, b_ref, out_ref):
    out_ref[...] = (
        jnp.sum(x_ref[...] * w_ref[...], axis=1, keepdims=True) + b_ref[0, 0]
    )


def _matvec(X, Wt, b2):
    N, D = X.shape
    BLK = 16384
    return pl.pallas_call(
        _mv_body,
        grid=(N // BLK,),
        in_specs=[
            pl.BlockSpec((BLK, D), lambda i: (i, 0)),
            pl.BlockSpec((1, D), lambda i: (0, 0)),
            pl.BlockSpec(memory_space=pltpu.SMEM),
        ],
        out_specs=pl.BlockSpec((BLK, 1), lambda i: (i, 0)),
        out_shape=jax.ShapeDtypeStruct((N, 1), jnp.float32),
    )(X, Wt, b2)


def _sc_fuse(bags2, mask2, nsent):
    nw, CHUNK = bags2.shape
    mesh = plsc.VectorSubcoreMesh(core_axis_name="c", subcore_axis_name="s")

    @functools.partial(
        pl.kernel,
        out_type=jax.ShapeDtypeStruct((nw, CHUNK), jnp.int32),
        mesh=mesh,
        scratch_types=[
            pltpu.VMEM((CHUNK,), jnp.int32),
            pltpu.VMEM((CHUNK,), jnp.int32),
        ],
    )
    def k(bags_hbm, mask_hbm, out_hbm, idx_v, msk_v):
        wid = lax.axis_index("s") * 2 + lax.axis_index("c")
        pltpu.sync_copy(bags_hbm.at[wid], idx_v)
        pltpu.sync_copy(mask_hbm.at[wid], msk_v)
        lane = lax.iota(jnp.int32, 16)

        def body(j, carry):
            off = j * 64
            for u in range(4):
                o = off + u * 16
                idx = idx_v[pl.ds(o, 16)]
                m = msk_v[pl.ds(o, 16)]
                pos = jnp.bitwise_and(o + lane, _PAD - 1)
                idx_v[pl.ds(o, 16)] = jnp.where(m != 0, nsent + pos, idx)
            return carry

        lax.fori_loop(0, CHUNK // 64, body, 0)
        pltpu.sync_copy(idx_v, out_hbm.at[wid])

    return k(bags2, mask2)


def _sc_gather_max(table, fused2, L):
    nw, CHUNK = fused2.shape
    bags_per_tile = CHUNK // L
    vregs_per_bag = L // 16
    mesh = plsc.VectorSubcoreMesh(core_axis_name="c", subcore_axis_name="s")
    oshape = jax.ShapeDtypeStruct((nw, bags_per_tile), jnp.float32)

    @functools.partial(
        pl.kernel,
        out_type=(oshape, oshape),
        mesh=mesh,
        scratch_types=[
            pltpu.VMEM((CHUNK,), jnp.int32),
            pltpu.VMEM((CHUNK,), jnp.float32),
            pltpu.VMEM((bags_per_tile,), jnp.float32),
            pltpu.VMEM((bags_per_tile,), jnp.float32),
            pltpu.SemaphoreType.DMA,
        ],
    )
    def k(table_hbm, fused_hbm, o0_hbm, o1_hbm, idx_v, vals_v, o0_v, o1_v, sem):
        wid = lax.axis_index("s") * 2 + lax.axis_index("c")
        pltpu.sync_copy(fused_hbm.at[wid], idx_v)
        pltpu.async_copy(table_hbm.at[idx_v], vals_v, sem).wait()
        neg = jnp.full((16,), -jnp.inf, dtype=jnp.float32)
        lane = lax.iota(jnp.int32, 16)

        dnums = lax.GatherDimensionNumbers(
            offset_dims=(), collapsed_slice_dims=(0,), start_index_map=(0,)
        )

        def vmax16(a):
            for s in (1, 2, 4, 8):
                perm = jnp.bitwise_xor(lane, s)
                shuf = lax.gather(
                    a, perm[:, None], dnums, (1,),
                    mode=lax.GatherScatterMode.PROMISE_IN_BOUNDS,
                )
                a = jnp.maximum(a, shuf)
            return a

        def group_body(g, carry):
            res = neg
            for k2 in range(16):
                off0 = (g * 16 + k2) * L
                acc = vals_v[pl.ds(off0, 16)]
                for i in range(1, vregs_per_bag):
                    acc = jnp.maximum(acc, vals_v[pl.ds(off0 + i * 16, 16)])
                res = jnp.where(lane == k2, vmax16(acc), res)
            t = jnp.exp(-jnp.abs(res))
            c = 1.0 + t
            y = 0.7 * t
            for _ in range(4):
                y = y + c * jnp.exp(-y) - 1.0
            sp = jnp.maximum(res, 0.0) + y
            o0_v[pl.ds(g * 16, 16)] = -sp
            o1_v[pl.ds(g * 16, 16)] = res - sp
            return carry

        lax.fori_loop(0, bags_per_tile // 16, group_body, 0)
        pltpu.sync_copy(o0_v, o0_hbm.at[wid])
        pltpu.sync_copy(o1_v, o1_hbm.at[wid])

    return k(table, fused2)


def kernel(X, bags, padding_mask, W, b):
    N, D = X.shape
    B, L = bags.shape
    bags2 = bags.astype(jnp.int32).reshape(_NW, (B * L) // _NW)
    mask2 = padding_mask.astype(jnp.int32).reshape(_NW, (B * L) // _NW)
    b_dep = b.reshape(1, 1) + (mask2[0, 0] * 0).astype(jnp.float32)
    logits = _matvec(X, W.reshape(1, D), b_dep)
    table = jnp.concatenate(
        [logits.reshape(N), jnp.full((_PAD,), -jnp.inf, jnp.float32)]
    )
    fused2 = _sc_fuse(bags2, mask2, N)
    o0, o1 = _sc_gather_max(table, fused2, L)
    return jnp.stack([o0.reshape(B), o1.reshape(B)], axis=-1)


# speedup vs baseline: 1.0022x; 1.0019x over previous
import functools

import jax
import jax.numpy as jnp
from jax import lax
from jax.experimental import pallas as pl
from jax.experimental.pallas import tpu as pltpu
from jax.experimental.pallas import tpu_sc as plsc

_NW = 32
_PAD = 16384


def _mv_body(x_ref, w_ref, b_ref, out_ref):
    out_ref[...] = (
        jnp.sum(x_ref[...] * w_ref[...], axis=1, keepdims=True) + b_ref[0, 0]
    )


def _matvec(X, Wt, b2):
    N, D = X.shape
    BLK = 16384
    return pl.pallas_call(
        _mv_body,
        grid=(N // BLK,),
        in_specs=[
            pl.BlockSpec((BLK, D), lambda i: (i, 0)),
            pl.BlockSpec((1, D), lambda i: (0, 0)),
            pl.BlockSpec(memory_space=pltpu.SMEM),
        ],
        out_specs=pl.BlockSpec((BLK, 1), lambda i: (i, 0)),
        out_shape=jax.ShapeDtypeStruct((N, 1), jnp.float32),
    )(X, Wt, b2)


def _sc_fuse(bags2, mask2, nsent):
    nw, CHUNK = bags2.shape
    mesh = plsc.VectorSubcoreMesh(core_axis_name="c", subcore_axis_name="s")

    @functools.partial(
        pl.kernel,
        out_type=jax.ShapeDtypeStruct((nw, CHUNK), jnp.int32),
        mesh=mesh,
        scratch_types=[
            pltpu.VMEM((CHUNK,), jnp.int32),
            pltpu.VMEM((CHUNK,), jnp.int32),
        ],
    )
    def k(bags_hbm, mask_hbm, out_hbm, idx_v, msk_v):
        wid = lax.axis_index("s") * 2 + lax.axis_index("c")
        pltpu.sync_copy(bags_hbm.at[wid], idx_v)
        pltpu.sync_copy(mask_hbm.at[wid], msk_v)
        lane = lax.iota(jnp.int32, 16)

        def body(j, carry):
            off = j * 64
            for u in range(4):
                o = off + u * 16
                idx = idx_v[pl.ds(o, 16)]
                m = msk_v[pl.ds(o, 16)]
                pos = jnp.bitwise_and(o + lane * 8, _PAD - 1)
                idx_v[pl.ds(o, 16)] = jnp.where(m != 0, nsent + pos, idx)
            return carry

        lax.fori_loop(0, CHUNK // 64, body, 0)
        pltpu.sync_copy(idx_v, out_hbm.at[wid])

    return k(bags2, mask2)


def _sc_gather_max(table, fused2, L):
    nw, CHUNK = fused2.shape
    bags_per_tile = CHUNK // L
    vregs_per_bag = L // 16
    mesh = plsc.VectorSubcoreMesh(core_axis_name="c", subcore_axis_name="s")
    oshape = jax.ShapeDtypeStruct((nw, bags_per_tile), jnp.float32)

    @functools.partial(
        pl.kernel,
        out_type=(oshape, oshape),
        mesh=mesh,
        scratch_types=[
            pltpu.VMEM((CHUNK,), jnp.int32),
            pltpu.VMEM((CHUNK,), jnp.float32),
            pltpu.VMEM((bags_per_tile,), jnp.float32),
            pltpu.VMEM((bags_per_tile,), jnp.float32),
            pltpu.SemaphoreType.DMA,
        ],
    )
    def k(table_hbm, fused_hbm, o0_hbm, o1_hbm, idx_v, vals_v, o0_v, o1_v, sem):
        wid = lax.axis_index("s") * 2 + lax.axis_index("c")
        pltpu.sync_copy(fused_hbm.at[wid], idx_v)
        pltpu.async_copy(table_hbm.at[idx_v], vals_v, sem).wait()
        neg = jnp.full((16,), -jnp.inf, dtype=jnp.float32)
        lane = lax.iota(jnp.int32, 16)

        dnums = lax.GatherDimensionNumbers(
            offset_dims=(), collapsed_slice_dims=(0,), start_index_map=(0,)
        )

        def vmax16(a):
            for s in (1, 2, 4, 8):
                perm = jnp.bitwise_xor(lane, s)
                shuf = lax.gather(
                    a, perm[:, None], dnums, (1,),
                    mode=lax.GatherScatterMode.PROMISE_IN_BOUNDS,
                )
                a = jnp.maximum(a, shuf)
            return a

        def group_body(g, carry):
            res = neg
            for k2 in range(16):
                off0 = (g * 16 + k2) * L
                acc = vals_v[pl.ds(off0, 16)]
                for i in range(1, vregs_per_bag):
                    acc = jnp.maximum(acc, vals_v[pl.ds(off0 + i * 16, 16)])
                res = jnp.where(lane == k2, vmax16(acc), res)
            t = jnp.exp(-jnp.abs(res))
            c = 1.0 + t
            y = 0.7 * t
            for _ in range(4):
                y = y + c * jnp.exp(-y) - 1.0
            sp = jnp.maximum(res, 0.0) + y
            o0_v[pl.ds(g * 16, 16)] = -sp
            o1_v[pl.ds(g * 16, 16)] = res - sp
            return carry

        lax.fori_loop(0, bags_per_tile // 16, group_body, 0)
        pltpu.sync_copy(o0_v, o0_hbm.at[wid])
        pltpu.sync_copy(o1_v, o1_hbm.at[wid])

    return k(table, fused2)


def kernel(X, bags, padding_mask, W, b):
    N, D = X.shape
    B, L = bags.shape
    bags2 = bags.astype(jnp.int32).reshape(_NW, (B * L) // _NW)
    mask2 = padding_mask.astype(jnp.int32).reshape(_NW, (B * L) // _NW)
    b_dep = b.reshape(1, 1) + (mask2[0, 0] * 0).astype(jnp.float32)
    logits = _matvec(X, W.reshape(1, D), b_dep)
    table = jnp.concatenate(
        [logits.reshape(N), jnp.full((_PAD,), -jnp.inf, jnp.float32)]
    )
    fused2 = _sc_fuse(bags2, mask2, N)
    o0, o1 = _sc_gather_max(table, fused2, L)
    return jnp.stack([o0.reshape(B), o1.reshape(B)], axis=-1)


# speedup vs baseline: 1.0607x; 1.0584x over previous
import functools

import jax
import jax.numpy as jnp
from jax import lax
from jax.experimental import pallas as pl
from jax.experimental.pallas import tpu as pltpu
from jax.experimental.pallas import tpu_sc as plsc

_NW = 32


def _mv_body(x_ref, w_ref, b_ref, out_ref):
    out_ref[...] = (
        jnp.sum(x_ref[...] * w_ref[...], axis=1, keepdims=True) + b_ref[0, 0]
    )


def _matvec(X, Wt, b2):
    N, D = X.shape
    BLK = 16384
    return pl.pallas_call(
        _mv_body,
        grid=(N // BLK,),
        in_specs=[
            pl.BlockSpec((BLK, D), lambda i: (i, 0)),
            pl.BlockSpec((1, D), lambda i: (0, 0)),
            pl.BlockSpec(memory_space=pltpu.SMEM),
        ],
        out_specs=pl.BlockSpec((BLK, 1), lambda i: (i, 0)),
        out_shape=jax.ShapeDtypeStruct((N, 1), jnp.float32),
    )(X, Wt, b2)


def _sc_gather_max(table, bags2, mask2, L):
    nw, CHUNK = bags2.shape
    bags_per_tile = CHUNK // L
    vregs_per_bag = L // 16
    mesh = plsc.VectorSubcoreMesh(core_axis_name="c", subcore_axis_name="s")
    oshape = jax.ShapeDtypeStruct((nw, bags_per_tile), jnp.float32)

    @functools.partial(
        pl.kernel,
        out_type=(oshape, oshape),
        mesh=mesh,
        scratch_types=[
            pltpu.VMEM((CHUNK,), jnp.int32),
            pltpu.VMEM((CHUNK,), jnp.int32),
            pltpu.VMEM((CHUNK,), jnp.float32),
            pltpu.VMEM((bags_per_tile,), jnp.float32),
            pltpu.VMEM((bags_per_tile,), jnp.float32),
            pltpu.SemaphoreType.DMA,
        ],
    )
    def k(table_hbm, bags_hbm, mask_hbm, o0_hbm, o1_hbm,
          idx_v, msk_v, vals_v, o0_v, o1_v, sem):
        wid = lax.axis_index("s") * 2 + lax.axis_index("c")
        pltpu.sync_copy(bags_hbm.at[wid], idx_v)
        cp = pltpu.async_copy(table_hbm.at[idx_v], vals_v, sem)
        pltpu.sync_copy(mask_hbm.at[wid], msk_v)
        cp.wait()
        neg = jnp.full((16,), -jnp.inf, dtype=jnp.float32)
        lane = lax.iota(jnp.int32, 16)

        dnums = lax.GatherDimensionNumbers(
            offset_dims=(), collapsed_slice_dims=(0,), start_index_map=(0,)
        )

        def vmax16(a):
            for s in (1, 2, 4, 8):
                perm = jnp.bitwise_xor(lane, s)
                shuf = lax.gather(
                    a, perm[:, None], dnums, (1,),
                    mode=lax.GatherScatterMode.PROMISE_IN_BOUNDS,
                )
                a = jnp.maximum(a, shuf)
            return a

        def group_body(g, carry):
            res = neg
            for k2 in range(16):
                off0 = (g * 16 + k2) * L
                acc = neg
                for i in range(vregs_per_bag):
                    off = off0 + i * 16
                    v = vals_v[pl.ds(off, 16)]
                    m = msk_v[pl.ds(off, 16)]
                    acc = jnp.maximum(acc, jnp.where(m != 0, -jnp.inf, v))
                res = jnp.where(lane == k2, vmax16(acc), res)
            t = jnp.exp(-jnp.abs(res))
            c = 1.0 + t
            y = 0.7 * t
            for _ in range(4):
                y = y + c * jnp.exp(-y) - 1.0
            sp = jnp.maximum(res, 0.0) + y
            o0_v[pl.ds(g * 16, 16)] = -sp
            o1_v[pl.ds(g * 16, 16)] = res - sp
            return carry

        lax.fori_loop(0, bags_per_tile // 16, group_body, 0)
        pltpu.sync_copy(o0_v, o0_hbm.at[wid])
        pltpu.sync_copy(o1_v, o1_hbm.at[wid])

    return k(table, bags2, mask2)


def kernel(X, bags, padding_mask, W, b):
    N, D = X.shape
    B, L = bags.shape
    logits = _matvec(X, W.reshape(1, D), b.reshape(1, 1))
    table = logits.reshape(N)
    bags2 = bags.astype(jnp.int32).reshape(_NW, (B * L) // _NW)
    mask2 = padding_mask.astype(jnp.int32).reshape(_NW, (B * L) // _NW)
    o0, o1 = _sc_gather_max(table, bags2, mask2, L)
    return jnp.stack([o0.reshape(B), o1.reshape(B)], axis=-1)
